# baseline (speedup 1.0000x reference)
import functools

import jax
import jax.numpy as jnp
from jax import lax
from jax.experimental import pallas as pl
from jax.experimental.pallas import tpu as pltpu
from jax.experimental.pallas import tpu_sc as plsc

PRE_SEQ_LEN = 128
HIDDEN = 1024
OUT_DIM = 24 * HIDDEN
BATCH = 16
N_TOK = BATCH * PRE_SEQ_LEN


_DT = 3072
_NT = OUT_DIM // _DT


def _mlp_body(e_ref, w1_ref, b1_ref, w2_ref, b2_ref, p_ref, h_ref):
    @pl.when(pl.program_id(0) == 0)
    def _():
        h = jnp.dot(e_ref[...], w1_ref[...], preferred_element_type=jnp.float32)
        h_ref[...] = jnp.tanh(h + b1_ref[...])

    p = jnp.dot(h_ref[...], w2_ref[...], preferred_element_type=jnp.float32)
    p_ref[...] = p + b2_ref[...]


def _mlp(emb_table, W1, b1, W2, b2):
    return pl.pallas_call(
        _mlp_body,
        grid=(_NT,),
        in_specs=[
            pl.BlockSpec((PRE_SEQ_LEN, HIDDEN), lambda j: (0, 0)),
            pl.BlockSpec((HIDDEN, HIDDEN), lambda j: (0, 0)),
            pl.BlockSpec((1, HIDDEN), lambda j: (0, 0)),
            pl.BlockSpec((HIDDEN, _DT), lambda j: (0, j)),
            pl.BlockSpec((1, _DT), lambda j: (0, j)),
        ],
        out_specs=pl.BlockSpec((PRE_SEQ_LEN, _DT), lambda j: (0, j)),
        out_shape=jax.ShapeDtypeStruct((PRE_SEQ_LEN, OUT_DIM), jnp.float32),
        scratch_shapes=[pltpu.VMEM((PRE_SEQ_LEN, HIDDEN), jnp.float32)],
    )(emb_table, W1, b1, W2, b2)


_NC = 2
_NS = 16
_NW = _NC * _NS
_SPLIT = 4
_SUB_DIM = OUT_DIM // _SPLIT
_N_SUB = N_TOK * _SPLIT
_B_PER_W = _N_SUB // _NW
_CH = 8


def _gather_body(p_hbm, idx_hbm, out_hbm, idx_v, rows_v, sem):
    wid = lax.axis_index("s") * _NC + lax.axis_index("c")
    base = wid * _B_PER_W
    pltpu.sync_copy(idx_hbm.at[pl.ds(base, _B_PER_W)], idx_v)

    def body(i, carry):
        pltpu.async_copy(
            p_hbm.at[idx_v.at[pl.ds(i * _CH, _CH)]], rows_v, sem
        ).wait()
        pltpu.sync_copy(rows_v, out_hbm.at[pl.ds(base + i * _CH, _CH)])
        return carry

    lax.fori_loop(0, _B_PER_W // _CH, body, 0)


@functools.cache
def _make_gather():
    return pl.kernel(
        _gather_body,
        out_type=jax.ShapeDtypeStruct((_N_SUB, _SUB_DIM), jnp.float32),
        mesh=plsc.VectorSubcoreMesh(
            core_axis_name="c", subcore_axis_name="s",
            num_cores=_NC, num_subcores=_NS,
        ),
        scratch_types=[
            pltpu.VMEM((_B_PER_W,), jnp.int32),
            pltpu.VMEM((_CH, _SUB_DIM), jnp.float32),
            pltpu.SemaphoreType.DMA,
        ],
    )


def kernel(prefix, emb_table, W1, b1, W2, b2):
    P = _mlp(emb_table, W1, b1.reshape(1, HIDDEN), W2, b2.reshape(1, OUT_DIM))
    idx = prefix.reshape(N_TOK).astype(jnp.int32)
    idx4 = (idx[:, None] * _SPLIT + jnp.arange(_SPLIT, dtype=jnp.int32)).reshape(_N_SUB)
    out = _make_gather()(P.reshape(_SPLIT * PRE_SEQ_LEN, _SUB_DIM), idx4)
    return out.reshape(BATCH, PRE_SEQ_LEN, OUT_DIM)


# speedup vs baseline: 1.0306x; 1.0306x over previous
import functools

import jax
import jax.numpy as jnp
from jax import lax
from jax.experimental import pallas as pl
from jax.experimental.pallas import tpu as pltpu
from jax.experimental.pallas import tpu_sc as plsc

PRE_SEQ_LEN = 128
HIDDEN = 1024
OUT_DIM = 24 * HIDDEN
BATCH = 16
N_TOK = BATCH * PRE_SEQ_LEN


_DT = 3072
_NT = OUT_DIM // _DT


def _mlp_body(e_ref, w1_ref, b1_ref, w2_ref, b2_ref, p_ref, h_ref):
    @pl.when(pl.program_id(0) == 0)
    def _():
        h = jnp.dot(e_ref[...], w1_ref[...], preferred_element_type=jnp.float32)
        h_ref[...] = jnp.tanh(h + b1_ref[...])

    p = jnp.dot(h_ref[...], w2_ref[...], preferred_element_type=jnp.float32)
    p_ref[...] = p + b2_ref[...]


def _mlp(emb_table, W1, b1, W2, b2):
    return pl.pallas_call(
        _mlp_body,
        grid=(_NT,),
        in_specs=[
            pl.BlockSpec((PRE_SEQ_LEN, HIDDEN), lambda j: (0, 0)),
            pl.BlockSpec((HIDDEN, HIDDEN), lambda j: (0, 0)),
            pl.BlockSpec((1, HIDDEN), lambda j: (0, 0)),
            pl.BlockSpec((HIDDEN, _DT), lambda j: (0, j)),
            pl.BlockSpec((1, _DT), lambda j: (0, j)),
        ],
        out_specs=pl.BlockSpec((PRE_SEQ_LEN, _DT), lambda j: (0, j)),
        out_shape=jax.ShapeDtypeStruct((PRE_SEQ_LEN, OUT_DIM), jnp.float32),
        scratch_shapes=[pltpu.VMEM((PRE_SEQ_LEN, HIDDEN), jnp.float32)],
    )(emb_table, W1, b1, W2, b2)


_NC = 2
_NS = 16
_NW = _NC * _NS
_SPLIT = 8
_SUB_DIM = OUT_DIM // _SPLIT
_N_SUB = N_TOK * _SPLIT
_B_PER_W = _N_SUB // _NW
_CH = 8
_NBUF = 4
_N_CHUNK = _B_PER_W // _CH


def _gather_body(p_hbm, idx_hbm, out_hbm, idx_v, rows_v, *sems):
    gsem = sems[:_NBUF]
    wsem = sems[_NBUF:]
    wid = lax.axis_index("s") * _NC + lax.axis_index("c")
    base = wid * _B_PER_W
    pltpu.sync_copy(idx_hbm.at[pl.ds(base, _B_PER_W)], idx_v)

    def g_start(c, b):
        pltpu.async_copy(
            p_hbm.at[idx_v.at[pl.ds(c * _CH, _CH)]], rows_v.at[b], gsem[b]
        )

    def w_start(c, b):
        pltpu.async_copy(
            rows_v.at[b], out_hbm.at[pl.ds(base + c * _CH, _CH)], wsem[b]
        )

    def g_wait(b):
        pltpu.make_async_copy(p_hbm.at[pl.ds(0, _CH)], rows_v.at[b], gsem[b]).wait()

    def w_wait(c, b):
        pltpu.make_async_copy(
            rows_v.at[b], out_hbm.at[pl.ds(base + c * _CH, _CH)], wsem[b]
        ).wait()

    for b in range(_NBUF):
        g_start(b, b)

    def outer(i, carry):
        g = i * _NBUF
        for b in range(_NBUF):
            c = g + b
            g_wait(b)
            w_start(c, b)
            w_wait(c, b)
            g_start(c + _NBUF, b)
        return carry

    lax.fori_loop(0, _N_CHUNK // _NBUF - 1, outer, 0)

    for b in range(_NBUF):
        c = _N_CHUNK - _NBUF + b
        g_wait(b)
        w_start(c, b)
    for b in range(_NBUF):
        c = _N_CHUNK - _NBUF + b
        w_wait(c, b)


@functools.cache
def _make_gather():
    return pl.kernel(
        _gather_body,
        out_type=jax.ShapeDtypeStruct((_N_SUB, _SUB_DIM), jnp.float32),
        mesh=plsc.VectorSubcoreMesh(
            core_axis_name="c", subcore_axis_name="s",
            num_cores=_NC, num_subcores=_NS,
        ),
        scratch_types=[
            pltpu.VMEM((_B_PER_W,), jnp.int32),
            pltpu.VMEM((_NBUF, _CH, _SUB_DIM), jnp.float32),
        ] + [pltpu.SemaphoreType.DMA] * (2 * _NBUF),
    )


def kernel(prefix, emb_table, W1, b1, W2, b2):
    P = _mlp(emb_table, W1, b1.reshape(1, HIDDEN), W2, b2.reshape(1, OUT_DIM))
    idx = prefix.reshape(N_TOK).astype(jnp.int32)
    idxs = (idx[:, None] * _SPLIT + jnp.arange(_SPLIT, dtype=jnp.int32)).reshape(_N_SUB)
    out = _make_gather()(P.reshape(_SPLIT * PRE_SEQ_LEN, _SUB_DIM), idxs)
    return out.reshape(BATCH, PRE_SEQ_LEN, OUT_DIM)


# speedup vs baseline: 2.1006x; 2.0383x over previous
import functools

import jax
import jax.numpy as jnp
from jax import lax
from jax.experimental import pallas as pl
from jax.experimental.pallas import tpu as pltpu
from jax.experimental.pallas import tpu_sc as plsc

PRE_SEQ_LEN = 128
HIDDEN = 1024
OUT_DIM = 24 * HIDDEN
BATCH = 16
N_TOK = BATCH * PRE_SEQ_LEN


_DT = 3072
_NT = OUT_DIM // _DT


def _mlp_body(e_ref, w1_ref, b1_ref, w2_ref, b2_ref, p_ref, h_ref):
    @pl.when(pl.program_id(0) == 0)
    def _():
        h = jnp.dot(e_ref[...], w1_ref[...], preferred_element_type=jnp.float32)
        h_ref[...] = jnp.tanh(h + b1_ref[...])

    p = jnp.dot(h_ref[...], w2_ref[...], preferred_element_type=jnp.float32)
    p_ref[...] = p + b2_ref[...]


def _mlp(emb_table, W1, b1, W2, b2):
    return pl.pallas_call(
        _mlp_body,
        grid=(_NT,),
        in_specs=[
            pl.BlockSpec((PRE_SEQ_LEN, HIDDEN), lambda j: (0, 0)),
            pl.BlockSpec((HIDDEN, HIDDEN), lambda j: (0, 0)),
            pl.BlockSpec((1, HIDDEN), lambda j: (0, 0)),
            pl.BlockSpec((HIDDEN, _DT), lambda j: (0, j)),
            pl.BlockSpec((1, _DT), lambda j: (0, j)),
        ],
        out_specs=pl.BlockSpec((PRE_SEQ_LEN, _DT), lambda j: (0, j)),
        out_shape=jax.ShapeDtypeStruct((PRE_SEQ_LEN, OUT_DIM), jnp.float32),
        scratch_shapes=[pltpu.VMEM((PRE_SEQ_LEN, HIDDEN), jnp.float32)],
    )(emb_table, W1, b1, W2, b2)


_NC = 2
_NS = 16
_NW = _NC * _NS
_TPW = N_TOK // _NW
_CH = 2
_NCHUNK = _TPW // _CH
_NBUF = 2


def _gather_body(p_hbm, idx_hbm, out_hbm, idx_v, rows_v, gsem, wsem):
    wid = lax.axis_index("s") * _NC + lax.axis_index("c")
    tok0 = wid * _TPW
    pltpu.sync_copy(idx_hbm.at[pl.ds(wid * _NCHUNK, _NCHUNK)], idx_v)

    def g_start(c, b):
        pltpu.async_copy(p_hbm.at[idx_v.at[c]], rows_v.at[b], gsem)

    def g_wait(b):
        pltpu.make_async_copy(p_hbm.at[idx_v.at[0]], rows_v.at[b], gsem).wait()

    def w_start(c, b):
        pltpu.async_copy(rows_v.at[b], out_hbm.at[pl.ds(tok0 + c * _CH, _CH)], wsem)

    def w_wait(c, b):
        pltpu.make_async_copy(
            rows_v.at[b], out_hbm.at[pl.ds(tok0 + c * _CH, _CH)], wsem
        ).wait()

    for b in range(_NBUF):
        g_start(b, b)

    def outer(i, carry):
        c = i * _NBUF
        for b in range(_NBUF):
            g_wait(b)
            w_start(c + b, b)
            w_wait(c + b, b)
            g_start(c + b + _NBUF, b)
        return carry

    lax.fori_loop(0, _NCHUNK // _NBUF - 1, outer, 0)

    for b in range(_NBUF):
        c = _NCHUNK - _NBUF + b
        g_wait(b)
        w_start(c, b)
    for b in range(_NBUF):
        w_wait(_NCHUNK - _NBUF + b, b)


@functools.cache
def _make_gather():
    return pl.kernel(
        _gather_body,
        out_type=jax.ShapeDtypeStruct((N_TOK, OUT_DIM), jnp.float32),
        mesh=plsc.VectorSubcoreMesh(
            core_axis_name="c", subcore_axis_name="s",
            num_cores=_NC, num_subcores=_NS,
        ),
        scratch_types=[
            pltpu.VMEM((_NCHUNK, _CH), jnp.int32),
            pltpu.VMEM((_NBUF, _CH, OUT_DIM), jnp.float32),
            pltpu.SemaphoreType.DMA,
            pltpu.SemaphoreType.DMA,
        ],
    )


def kernel(prefix, emb_table, W1, b1, W2, b2):
    P = _mlp(emb_table, W1, b1.reshape(1, HIDDEN), W2, b2.reshape(1, OUT_DIM))
    idx = prefix.reshape(N_TOK).astype(jnp.int32)
    out = _make_gather()(P, idx.reshape(N_TOK // _CH, _CH))
    return out.reshape(BATCH, PRE_SEQ_LEN, OUT_DIM)
